# baseline (speedup 1.0000x reference)
import jax
import jax.numpy as jnp
from jax.experimental import pallas as pl


def _conv_body(x_ref, ah_ref, am_ref, w1_ref, b1_ref, w2_ref, b2_ref,
               w3_ref, b3_ref, wc_ref, bc_ref, xc_ref, aout_ref):
    f32 = jnp.float32
    h = ah_ref[0]
    h = jnp.maximum(jnp.dot(w1_ref[...], h, preferred_element_type=f32)
                    + b1_ref[...], 0.0)
    h = jnp.maximum(jnp.dot(w2_ref[...], h, preferred_element_type=f32)
                    + b2_ref[...], 0.0)
    h = jnp.maximum(jnp.dot(w3_ref[...], h, preferred_element_type=f32)
                    + b3_ref[...], 0.0)
    aout_ref[0] = h * am_ref[0]
    xc_ref[0] = (jnp.dot(wc_ref[...], x_ref[0], preferred_element_type=f32)
                 + bc_ref[...])


def _bmm_body(xc_ref, a_ref, out_ref):
    for j in range(xc_ref.shape[1]):
        out_ref[0, j] = jnp.dot(xc_ref[0, j], a_ref[0, j],
                                preferred_element_type=jnp.float32)


def kernel(x, A, W1, b1, W2, b2, W3, b3, Wc, bc):
    n, c, t, v = x.shape
    k = A.shape[1]
    vv = v * v
    tv = t * v

    x_flat = x.reshape(n, c, tv)
    a_h = A[:, :k - 1].reshape(n, k - 1, vv)
    a_m = A[:, k - 1:].reshape(n, 1, vv)
    b1c = b1.reshape(-1, 1)
    b2c = b2.reshape(-1, 1)
    b3c = b3.reshape(-1, 1)
    bcc = bc.reshape(-1, 1)

    full = lambda a: pl.BlockSpec(a.shape, lambda i: (0,) * a.ndim)

    xc_flat, aout_flat = pl.pallas_call(
        _conv_body,
        grid=(n,),
        in_specs=[
            pl.BlockSpec((1, c, tv), lambda i: (i, 0, 0)),
            pl.BlockSpec((1, k - 1, vv), lambda i: (i, 0, 0)),
            pl.BlockSpec((1, 1, vv), lambda i: (i, 0, 0)),
            full(W1), full(b1c), full(W2), full(b2c),
            full(W3), full(b3c), full(Wc), full(bcc),
        ],
        out_specs=[
            pl.BlockSpec((1, c, tv), lambda i: (i, 0, 0)),
            pl.BlockSpec((1, c, vv), lambda i: (i, 0, 0)),
        ],
        out_shape=[
            jax.ShapeDtypeStruct((n, c, tv), jnp.float32),
            jax.ShapeDtypeStruct((n, c, vv), jnp.float32),
        ],
    )(x_flat, a_h, a_m, W1, b1c, W2, b2c, W3, b3c, Wc, bcc)

    xc = xc_flat.reshape(n, c, t, v)
    aout = aout_flat.reshape(n, c, v, v)

    cb = 8
    out = pl.pallas_call(
        _bmm_body,
        grid=(n, c // cb),
        in_specs=[
            pl.BlockSpec((1, cb, t, v), lambda i, j: (i, j, 0, 0)),
            pl.BlockSpec((1, cb, v, v), lambda i, j: (i, j, 0, 0)),
        ],
        out_specs=pl.BlockSpec((1, cb, t, v), lambda i, j: (i, j, 0, 0)),
        out_shape=jax.ShapeDtypeStruct((n, c, t, v), jnp.float32),
    )(xc, aout)

    return (out, aout)


# speedup vs baseline: 1.6662x; 1.6662x over previous
import jax
import jax.numpy as jnp
from jax.experimental import pallas as pl


def _fused_body(x_ref, ah_ref, am_ref, w1_ref, b1_ref, w2_ref, b2_ref,
                w3_ref, b3_ref, wc_ref, bc_ref, out_ref, aout_ref):
    f32 = jnp.float32
    c, t, v = out_ref.shape[1], out_ref.shape[2], out_ref.shape[3]

    h = ah_ref[0]
    h = jnp.maximum(jnp.dot(w1_ref[...], h, preferred_element_type=f32)
                    + b1_ref[...], 0.0)
    h = jnp.maximum(jnp.dot(w2_ref[...], h, preferred_element_type=f32)
                    + b2_ref[...], 0.0)
    h = jnp.maximum(jnp.dot(w3_ref[...], h, preferred_element_type=f32)
                    + b3_ref[...], 0.0)
    aout2 = h * am_ref[0]
    aout_ref[0] = aout2

    xc2 = (jnp.dot(wc_ref[...], x_ref[0], preferred_element_type=f32)
           + bc_ref[...])
    xc3 = xc2.reshape(c, t, v)
    aout3 = aout2.reshape(c, v, v)
    for j in range(c):
        out_ref[0, j] = jnp.dot(xc3[j], aout3[j],
                                preferred_element_type=f32)


def kernel(x, A, W1, b1, W2, b2, W3, b3, Wc, bc):
    n, c, t, v = x.shape
    k = A.shape[1]
    vv = v * v
    tv = t * v

    x_flat = x.reshape(n, c, tv)
    a_h = A[:, :k - 1].reshape(n, k - 1, vv)
    a_m = A[:, k - 1:].reshape(n, 1, vv)
    b1c = b1.reshape(-1, 1)
    b2c = b2.reshape(-1, 1)
    b3c = b3.reshape(-1, 1)
    bcc = bc.reshape(-1, 1)

    full = lambda a: pl.BlockSpec(a.shape, lambda i: (0,) * a.ndim)

    out, aout_flat = pl.pallas_call(
        _fused_body,
        grid=(n,),
        in_specs=[
            pl.BlockSpec((1, c, tv), lambda i: (i, 0, 0)),
            pl.BlockSpec((1, k - 1, vv), lambda i: (i, 0, 0)),
            pl.BlockSpec((1, 1, vv), lambda i: (i, 0, 0)),
            full(W1), full(b1c), full(W2), full(b2c),
            full(W3), full(b3c), full(Wc), full(bcc),
        ],
        out_specs=[
            pl.BlockSpec((1, c, t, v), lambda i: (i, 0, 0, 0)),
            pl.BlockSpec((1, c, vv), lambda i: (i, 0, 0)),
        ],
        out_shape=[
            jax.ShapeDtypeStruct((n, c, t, v), jnp.float32),
            jax.ShapeDtypeStruct((n, c, vv), jnp.float32),
        ],
    )(x_flat, a_h, a_m, W1, b1c, W2, b2c, W3, b3c, Wc, bcc)

    return (out, aout_flat.reshape(n, c, v, v))


# speedup vs baseline: 1.6687x; 1.0015x over previous
import jax
import jax.numpy as jnp
from jax.experimental import pallas as pl


def _fused_body(x_ref, a_ref, w1_ref, b1_ref, w2_ref, b2_ref,
                w3_ref, b3_ref, wc_ref, bc_ref, out_ref, aout_ref):
    f32 = jnp.float32
    c, t, v = out_ref.shape[1], out_ref.shape[2], out_ref.shape[3]
    k = a_ref.shape[1]

    h = a_ref[0, :k - 1]
    h = jnp.maximum(jnp.dot(w1_ref[...], h, preferred_element_type=f32)
                    + b1_ref[...], 0.0)
    h = jnp.maximum(jnp.dot(w2_ref[...], h, preferred_element_type=f32)
                    + b2_ref[...], 0.0)
    h = jnp.maximum(jnp.dot(w3_ref[...], h, preferred_element_type=f32)
                    + b3_ref[...], 0.0)
    aout2 = h * a_ref[0, k - 1:]
    aout_ref[0] = aout2

    xc2 = (jnp.dot(wc_ref[...], x_ref[0], preferred_element_type=f32)
           + bc_ref[...])
    xc3 = xc2.reshape(c, t, v)
    aout3 = aout2.reshape(c, v, v)
    for j in range(c):
        out_ref[0, j] = jnp.dot(xc3[j], aout3[j],
                                preferred_element_type=f32)


def kernel(x, A, W1, b1, W2, b2, W3, b3, Wc, bc):
    n, c, t, v = x.shape
    k = A.shape[1]
    vv = v * v
    tv = t * v

    x_flat = x.reshape(n, c, tv)
    a_flat = A.reshape(n, k, vv)
    b1c = b1.reshape(-1, 1)
    b2c = b2.reshape(-1, 1)
    b3c = b3.reshape(-1, 1)
    bcc = bc.reshape(-1, 1)

    full = lambda a: pl.BlockSpec(a.shape, lambda i: (0,) * a.ndim)

    out, aout_flat = pl.pallas_call(
        _fused_body,
        grid=(n,),
        in_specs=[
            pl.BlockSpec((1, c, tv), lambda i: (i, 0, 0)),
            pl.BlockSpec((1, k, vv), lambda i: (i, 0, 0)),
            full(W1), full(b1c), full(W2), full(b2c),
            full(W3), full(b3c), full(Wc), full(bcc),
        ],
        out_specs=[
            pl.BlockSpec((1, c, t, v), lambda i: (i, 0, 0, 0)),
            pl.BlockSpec((1, c, vv), lambda i: (i, 0, 0)),
        ],
        out_shape=[
            jax.ShapeDtypeStruct((n, c, t, v), jnp.float32),
            jax.ShapeDtypeStruct((n, c, vv), jnp.float32),
        ],
    )(x_flat, a_flat, W1, b1c, W2, b2c, W3, b3c, Wc, bcc)

    return (out, aout_flat.reshape(n, c, v, v))


# speedup vs baseline: 3.0516x; 1.8287x over previous
import jax
import jax.numpy as jnp
from jax.experimental import pallas as pl


def _fused_body(x_ref, a_ref, w1_ref, b1_ref, w2_ref, b2_ref,
                w3_ref, b3_ref, wc_ref, bc_ref, out_ref, aout_ref):
    f32 = jnp.float32
    c, w, t = out_ref.shape[1], out_ref.shape[2], out_ref.shape[3]
    v = w
    k = a_ref.shape[1]

    h = a_ref[0, :k - 1]
    h = jnp.maximum(jnp.dot(w1_ref[...], h, preferred_element_type=f32)
                    + b1_ref[...], 0.0)
    h = jnp.maximum(jnp.dot(w2_ref[...], h, preferred_element_type=f32)
                    + b2_ref[...], 0.0)
    h = jnp.maximum(jnp.dot(w3_ref[...], h, preferred_element_type=f32)
                    + b3_ref[...], 0.0)
    aout2 = h * a_ref[0, k - 1:]
    aout_ref[0] = aout2

    xc2 = (jnp.dot(wc_ref[...], x_ref[0], preferred_element_type=f32)
           + bc_ref[...])
    xc3 = xc2.reshape(c, v, t)
    aout3 = aout2.reshape(c, v, w)
    for j in range(c):
        out_ref[0, j] = jax.lax.dot_general(
            aout3[j], xc3[j], (((0,), (0,)), ((), ())),
            preferred_element_type=f32)


def kernel(x, A, W1, b1, W2, b2, W3, b3, Wc, bc):
    n, c, t, v = x.shape
    k = A.shape[1]
    vv = v * v
    vt = v * t

    xt_flat = jnp.swapaxes(x, 2, 3).reshape(n, c, vt)
    a_flat = A.reshape(n, k, vv)
    b1c = b1.reshape(-1, 1)
    b2c = b2.reshape(-1, 1)
    b3c = b3.reshape(-1, 1)
    bcc = bc.reshape(-1, 1)

    full = lambda a: pl.BlockSpec(a.shape, lambda i: (0,) * a.ndim)

    outt, aout_flat = pl.pallas_call(
        _fused_body,
        grid=(n,),
        in_specs=[
            pl.BlockSpec((1, c, vt), lambda i: (i, 0, 0)),
            pl.BlockSpec((1, k, vv), lambda i: (i, 0, 0)),
            full(W1), full(b1c), full(W2), full(b2c),
            full(W3), full(b3c), full(Wc), full(bcc),
        ],
        out_specs=[
            pl.BlockSpec((1, c, v, t), lambda i: (i, 0, 0, 0)),
            pl.BlockSpec((1, c, vv), lambda i: (i, 0, 0)),
        ],
        out_shape=[
            jax.ShapeDtypeStruct((n, c, v, t), jnp.float32),
            jax.ShapeDtypeStruct((n, c, vv), jnp.float32),
        ],
    )(xt_flat, a_flat, W1, b1c, W2, b2c, W3, b3c, Wc, bcc)

    out = jnp.swapaxes(outt, 2, 3)
    return (out, aout_flat.reshape(n, c, v, v))


# speedup vs baseline: 5.3965x; 1.7684x over previous
import jax
import jax.numpy as jnp
from jax.experimental import pallas as pl


def _cdot(w, h3):
    return jax.lax.dot_general(w, h3, (((1,), (0,)), ((), ())),
                               preferred_element_type=jnp.float32)


def _fused_body(x_ref, a_ref, w1_ref, b1_ref, w2_ref, b2_ref,
                w3_ref, b3_ref, wc_ref, bc_ref, out_ref, aout_ref):
    f32 = jnp.float32
    c = out_ref.shape[1]
    k = a_ref.shape[1]

    h = a_ref[0, :k - 1]
    h = jnp.maximum(_cdot(w1_ref[...], h) + b1_ref[...][:, :, None], 0.0)
    h = jnp.maximum(_cdot(w2_ref[...], h) + b2_ref[...][:, :, None], 0.0)
    h = jnp.maximum(_cdot(w3_ref[...], h) + b3_ref[...][:, :, None], 0.0)
    aout3 = h * a_ref[0, k - 1:]
    aout_ref[0] = aout3

    xc3 = _cdot(wc_ref[...], x_ref[0]) + bc_ref[...][:, :, None]
    for j in range(c):
        out_ref[0, j] = jax.lax.dot_general(
            aout3[j], xc3[j], (((0,), (0,)), ((), ())),
            preferred_element_type=f32)


def kernel(x, A, W1, b1, W2, b2, W3, b3, Wc, bc):
    n, c, t, v = x.shape
    k = A.shape[1]

    xt = jnp.swapaxes(x, 2, 3)
    b1c = b1.reshape(-1, 1)
    b2c = b2.reshape(-1, 1)
    b3c = b3.reshape(-1, 1)
    bcc = bc.reshape(-1, 1)

    full = lambda a: pl.BlockSpec(a.shape, lambda i: (0,) * a.ndim)

    outt, aout = pl.pallas_call(
        _fused_body,
        grid=(n,),
        in_specs=[
            pl.BlockSpec((1, c, v, t), lambda i: (i, 0, 0, 0)),
            pl.BlockSpec((1, k, v, v), lambda i: (i, 0, 0, 0)),
            full(W1), full(b1c), full(W2), full(b2c),
            full(W3), full(b3c), full(Wc), full(bcc),
        ],
        out_specs=[
            pl.BlockSpec((1, c, v, t), lambda i: (i, 0, 0, 0)),
            pl.BlockSpec((1, c, v, v), lambda i: (i, 0, 0, 0)),
        ],
        out_shape=[
            jax.ShapeDtypeStruct((n, c, v, t), jnp.float32),
            jax.ShapeDtypeStruct((n, c, v, v), jnp.float32),
        ],
    )(xt, A, W1, b1c, W2, b2c, W3, b3c, Wc, bcc)

    out = jnp.swapaxes(outt, 2, 3)
    return (out, aout)


# speedup vs baseline: 5.7135x; 1.0587x over previous
import jax
import jax.numpy as jnp
from jax.experimental import pallas as pl


def _cdot(wt, h3):
    return jax.lax.dot_general(wt, h3, (((0,), (0,)), ((), ())),
                               preferred_element_type=jnp.float32)


def _cdot_r(w, h3):
    return jax.lax.dot_general(w, h3, (((1,), (0,)), ((), ())),
                               preferred_element_type=jnp.float32)


def _fused_body(x_ref, a_ref, w1_ref, b1_ref, w2_ref, b2_ref,
                w3_ref, b3_ref, wc_ref, bc_ref, out_ref, aout_ref):
    f32 = jnp.float32
    c = out_ref.shape[1]
    k = a_ref.shape[1]

    h = a_ref[0, :k - 1]
    h = jnp.maximum(_cdot(w1_ref[...], h) + b1_ref[...][:, :, None], 0.0)
    h = jnp.maximum(_cdot(w2_ref[...], h) + b2_ref[...][:, :, None], 0.0)
    h = jnp.maximum(_cdot(w3_ref[...], h) + b3_ref[...][:, :, None], 0.0)
    aout3 = h * a_ref[0, k - 1:]
    aout_ref[0] = aout3

    xc3 = _cdot_r(wc_ref[...], x_ref[0]) + bc_ref[...][:, :, None]
    for j in range(c):
        out_ref[0, j] = jax.lax.dot_general(
            aout3[j], xc3[j], (((0,), (0,)), ((), ())),
            preferred_element_type=f32)


def kernel(x, A, W1, b1, W2, b2, W3, b3, Wc, bc):
    n, c, t, v = x.shape
    k = A.shape[1]

    xt = jnp.swapaxes(x, 2, 3)
    w1t = jnp.swapaxes(W1, 0, 1)
    w2t = jnp.swapaxes(W2, 0, 1)
    w3t = jnp.swapaxes(W3, 0, 1)
    b1c = b1.reshape(-1, 1)
    b2c = b2.reshape(-1, 1)
    b3c = b3.reshape(-1, 1)
    bcc = bc.reshape(-1, 1)

    full = lambda a: pl.BlockSpec(a.shape, lambda i: (0,) * a.ndim)

    outt, aout = pl.pallas_call(
        _fused_body,
        grid=(n,),
        in_specs=[
            pl.BlockSpec((1, c, v, t), lambda i: (i, 0, 0, 0)),
            pl.BlockSpec((1, k, v, v), lambda i: (i, 0, 0, 0)),
            full(w1t), full(b1c), full(w2t), full(b2c),
            full(w3t), full(b3c), full(Wc), full(bcc),
        ],
        out_specs=[
            pl.BlockSpec((1, c, v, t), lambda i: (i, 0, 0, 0)),
            pl.BlockSpec((1, c, v, v), lambda i: (i, 0, 0, 0)),
        ],
        out_shape=[
            jax.ShapeDtypeStruct((n, c, v, t), jnp.float32),
            jax.ShapeDtypeStruct((n, c, v, v), jnp.float32),
        ],
    )(xt, A, w1t, b1c, w2t, b2c, w3t, b3c, Wc, bcc)

    out = jnp.swapaxes(outt, 2, 3)
    return (out, aout)
